# baseline (speedup 1.0000x reference)
import functools

import jax
import jax.numpy as jnp
from jax import lax
from jax.experimental import pallas as pl
from jax.experimental.pallas import tpu as pltpu
from jax.experimental.pallas import tpu_sc as plsc

B = 128
F = 17
H = 128
W = 128
B_SC = 96
B_TC = B - B_SC
RCH = 16
NCH = H // RCH
NW = 32
BPW = B_SC // NW
NCHG = BPW * NCH
NVEC = RCH * W // 16
HIST = BPW * F * 16

THRESH_KEY = 0x3F000000


def _sc_call(x):
    mesh = plsc.VectorSubcoreMesh(core_axis_name="c", subcore_axis_name="s")

    @functools.partial(
        pl.kernel,
        mesh=mesh,
        compiler_params=pltpu.CompilerParams(needs_layout_passes=False),
        out_type=jax.ShapeDtypeStruct((NW * HIST,), jnp.float32),
        scratch_types=[
            pltpu.VMEM((F, RCH, W), jnp.float32),
            pltpu.VMEM((F, RCH, W), jnp.float32),
            pltpu.VMEM((HIST,), jnp.float32),
            pltpu.SemaphoreType.DMA,
            pltpu.SemaphoreType.DMA,
        ],
    )
    def run(x_hbm, out_hbm, buf0, buf1, hist, sem0, sem1):
        wid = lax.axis_index("c") * 16 + lax.axis_index("s")
        lane = lax.broadcasted_iota(jnp.int32, (16,), 0)
        zeros = jnp.zeros((16,), jnp.float32)
        ones = jnp.ones((16,), jnp.float32)
        bufs = (buf0, buf1)
        sems = (sem0, sem1)

        def dma(g, p):
            b = B_TC + wid * BPW + (g >> 3)
            h0 = (g & 7) * RCH
            return pltpu.make_async_copy(
                x_hbm.at[b, :, pl.ds(h0, RCH), :], bufs[p], sems[p]
            )

        dma(0, 0).start()

        def zero_body(i, c):
            hist[pl.ds(i * 16, 16)] = zeros
            return c

        lax.fori_loop(0, HIST // 16, zero_body, 0)

        def outer(t, c):
            for p in range(2):
                g = t * 2 + p
                dma(g, p).wait()

                @pl.when(g < NCHG - 1)
                def _(g=g, p=p):
                    dma(g + 1, 1 - p).start()

                buf = bufs[p]
                cvec = lane + ((g >> 3) * (F * 16) + 256)

                @plsc.parallel_loop(0, NVEC, unroll=4)
                def vec(i, buf=buf, cvec=cvec):
                    r = i >> 3
                    w0 = (i & 7) * 16
                    k = (
                        lax.bitcast_convert_type(
                            buf[0, r, pl.ds(w0, 16)], jnp.uint32
                        )
                        & ~jnp.uint32(31)
                    ) | jnp.uint32(16)
                    for cc in range(1, F):
                        kc = (
                            lax.bitcast_convert_type(
                                buf[cc, r, pl.ds(w0, 16)], jnp.uint32
                            )
                            & ~jnp.uint32(31)
                        ) | jnp.uint32(F - 1 - cc)
                        k = jnp.maximum(k, kc)
                    ki = lax.bitcast_convert_type(k, jnp.int32)
                    idx = cvec - ((ki & 31) << 4)
                    valid = ki >= THRESH_KEY
                    plsc.addupdate_scatter(hist, [idx], ones, mask=valid)

            return c

        lax.fori_loop(0, NCHG // 2, outer, 0)

        pltpu.sync_copy(hist, out_hbm.at[pl.ds(wid * HIST, HIST)])

    return run(x)


def _tc_body(x_ref, o_ref):
    bits = lax.bitcast_convert_type(x_ref[0], jnp.int32)
    m = jnp.max(bits, axis=0)
    m = jnp.where(m >= jnp.int32(THRESH_KEY), m, jnp.int32(-1))
    eq = (bits == m[None]).astype(jnp.float32)
    o_ref[pl.program_id(0)] = jnp.sum(eq, axis=(1, 2))


def _tc_call(x):
    return pl.pallas_call(
        _tc_body,
        grid=(B_TC,),
        in_specs=[pl.BlockSpec((1, F, H, W), lambda i: (i, 0, 0, 0))],
        out_specs=pl.BlockSpec((B_TC, F), lambda i: (0, 0)),
        out_shape=jax.ShapeDtypeStruct((B_TC, F), jnp.float32),
    )(x)


def kernel(x):
    sc = _sc_call(x).reshape(B_SC, F, 16).sum(axis=-1)
    tc = _tc_call(x)
    return jnp.concatenate([tc, sc], axis=0)


# speedup vs baseline: 1.1209x; 1.1209x over previous
import functools

import jax
import jax.numpy as jnp
from jax import lax
from jax.experimental import pallas as pl
from jax.experimental.pallas import tpu as pltpu
from jax.experimental.pallas import tpu_sc as plsc

B = 128
F = 17
H = 128
W = 128
B_SC = 96
B_TC = B - B_SC
RCH = 16
NCH = H // RCH
NW = 32
BPW = B_SC // NW
NCHG = BPW * NCH
NVEC = RCH * W // 16
HIST = BPW * F * 16

THRESH_KEY = 0x3F000000


def _sc_call(x):
    mesh = plsc.VectorSubcoreMesh(core_axis_name="c", subcore_axis_name="s")

    @functools.partial(
        pl.kernel,
        mesh=mesh,
        compiler_params=pltpu.CompilerParams(needs_layout_passes=False),
        out_type=jax.ShapeDtypeStruct((NW * HIST,), jnp.float32),
        scratch_types=[
            pltpu.VMEM((F, RCH, W), jnp.float32),
            pltpu.VMEM((F, RCH, W), jnp.float32),
            pltpu.VMEM((F, RCH, W), jnp.float32),
            pltpu.VMEM((HIST,), jnp.float32),
            pltpu.SemaphoreType.DMA,
            pltpu.SemaphoreType.DMA,
            pltpu.SemaphoreType.DMA,
        ],
    )
    def run(x_hbm, out_hbm, buf0, buf1, buf2, hist, sem0, sem1, sem2):
        wid = lax.axis_index("c") * 16 + lax.axis_index("s")
        lane = lax.broadcasted_iota(jnp.int32, (16,), 0)
        zeros = jnp.zeros((16,), jnp.float32)
        ones = jnp.ones((16,), jnp.float32)
        bufs = (buf0, buf1, buf2)
        sems = (sem0, sem1, sem2)

        def dma(g, p):
            b = B_TC + wid * BPW + (g >> 3)
            h0 = (g & 7) * RCH
            return pltpu.make_async_copy(
                x_hbm.at[b, :, pl.ds(h0, RCH), :], bufs[p], sems[p]
            )

        dma(0, 0).start()
        dma(1, 1).start()

        def zero_body(i, c):
            hist[pl.ds(i * 16, 16)] = zeros
            return c

        lax.fori_loop(0, HIST // 16, zero_body, 0)

        def outer(t, c):
            for p in range(3):
                g = t * 3 + p
                dma(g, p).wait()

                @pl.when(g < NCHG - 2)
                def _(g=g, p=(p + 2) % 3):
                    dma(g + 2, p).start()

                buf = bufs[p]
                cvec = lane + ((g >> 3) * (F * 16) + 256)

                @plsc.parallel_loop(0, NVEC, unroll=4)
                def vec(i, buf=buf, cvec=cvec):
                    r = i >> 3
                    w0 = (i & 7) * 16
                    k = (
                        lax.bitcast_convert_type(
                            buf[0, r, pl.ds(w0, 16)], jnp.uint32
                        )
                        & ~jnp.uint32(31)
                    ) | jnp.uint32(16)
                    for cc in range(1, F):
                        kc = (
                            lax.bitcast_convert_type(
                                buf[cc, r, pl.ds(w0, 16)], jnp.uint32
                            )
                            & ~jnp.uint32(31)
                        ) | jnp.uint32(F - 1 - cc)
                        k = jnp.maximum(k, kc)
                    ki = lax.bitcast_convert_type(k, jnp.int32)
                    idx = cvec - ((ki & 31) << 4)
                    valid = ki >= THRESH_KEY
                    plsc.addupdate_scatter(hist, [idx], ones, mask=valid)

            return c

        lax.fori_loop(0, NCHG // 3, outer, 0)

        pltpu.sync_copy(hist, out_hbm.at[pl.ds(wid * HIST, HIST)])

    return run(x)


def _tc_body(x_ref, o_ref):
    bits = lax.bitcast_convert_type(x_ref[0], jnp.int32)
    m = jnp.max(bits, axis=0)
    m = jnp.where(m >= jnp.int32(THRESH_KEY), m, jnp.int32(-1))
    eq = (bits == m[None]).astype(jnp.float32)
    o_ref[pl.program_id(0)] = jnp.sum(eq, axis=(1, 2))


def _tc_call(x):
    return pl.pallas_call(
        _tc_body,
        grid=(B_TC,),
        in_specs=[pl.BlockSpec((1, F, H, W), lambda i: (i, 0, 0, 0))],
        out_specs=pl.BlockSpec((B_TC, F), lambda i: (0, 0)),
        out_shape=jax.ShapeDtypeStruct((B_TC, F), jnp.float32),
    )(x)


def kernel(x):
    sc = _sc_call(x).reshape(B_SC, F, 16).sum(axis=-1)
    tc = _tc_call(x)
    return jnp.concatenate([tc, sc], axis=0)
